# baseline (speedup 1.0000x reference)
import functools

import jax
import jax.numpy as jnp
from jax import lax
from jax.experimental import pallas as pl
from jax.experimental.pallas import tpu as pltpu
from jax.experimental.pallas import tpu_sc as plsc

NC = 2
NS = 16
NW = NC * NS
LANES = 16
CH = 40


@functools.lru_cache(maxsize=None)
def _build(B, L, V, E):
    assert B % NW == 0 and L % CH == 0 and E % LANES == 0
    rows_w = B // NW
    halves = L // CH
    g_total = rows_w * halves

    mesh = plsc.VectorSubcoreMesh(
        core_axis_name="c", subcore_axis_name="s", num_cores=NC, num_subcores=NS
    )

    def body(x_hbm, tok_hbm, pos_hbm, out_hbm,
             idx_v, pos_v, gb0, gb1, ob0, ob1, gs0, gs1, ss0, ss1):
        gbufs = (gb0, gb1)
        obufs = (ob0, ob1)
        gsems = (gs0, gs1)
        ssems = (ss0, ss1)
        wid = lax.axis_index("s") * NC + lax.axis_index("c")
        seq0 = wid * rows_w

        pltpu.sync_copy(x_hbm.at[pl.ds(seq0, rows_w)], idx_v)
        pltpu.sync_copy(pos_hbm, pos_v)

        def start_gather(g):
            b = g % 2
            idx = idx_v.at[g // halves, pl.ds((g % halves) * CH, CH)]
            return pltpu.async_copy(tok_hbm.at[idx], gbufs[b], gsems[b])

        gathers = {0: start_gather(0), 1: start_gather(1)}
        scatters = {}

        for g in range(g_total):
            b = g % 2
            gathers.pop(g).wait()
            if g >= 2:
                scatters.pop(g - 2).wait()
            phase = (g % halves) * CH
            gb, ob = gbufs[b], obufs[b]

            def add_row(i, _, gb=gb, ob=ob, phase=phase):
                for j in range(E // LANES):
                    c = j * LANES
                    ob[i, pl.ds(c, LANES)] = (
                        gb[i, pl.ds(c, LANES)] + pos_v[phase + i, pl.ds(c, LANES)]
                    )
                return 0

            lax.fori_loop(0, CH, add_row, 0)

            if g + 2 < g_total:
                gathers[g + 2] = start_gather(g + 2)
            scatters[g] = pltpu.async_copy(
                obufs[b],
                out_hbm.at[seq0 + g // halves, pl.ds((g % halves) * CH, CH)],
                ssems[b],
            )

        scatters.pop(g_total - 2).wait()
        scatters.pop(g_total - 1).wait()

    return pl.kernel(
        body,
        out_type=jax.ShapeDtypeStruct((B, L, E), jnp.float32),
        mesh=mesh,
        compiler_params=pltpu.CompilerParams(use_tc_tiling_on_sc=False),
        scratch_types=[
            pltpu.VMEM((B // NW, L), jnp.int32),
            pltpu.VMEM((L, E), jnp.float32),
            pltpu.VMEM((CH, E), jnp.float32),
            pltpu.VMEM((CH, E), jnp.float32),
            pltpu.VMEM((CH, E), jnp.float32),
            pltpu.VMEM((CH, E), jnp.float32),
            pltpu.SemaphoreType.DMA,
            pltpu.SemaphoreType.DMA,
            pltpu.SemaphoreType.DMA,
            pltpu.SemaphoreType.DMA,
        ],
    )


def kernel(x, token_table, pos_table):
    B, L = x.shape
    V, E = token_table.shape
    k = _build(B, L, V, E)
    return k(x.astype(jnp.int32), token_table, pos_table)


# speedup vs baseline: 1.1446x; 1.1446x over previous
import functools

import jax
import jax.numpy as jnp
from jax import lax
from jax.experimental import pallas as pl
from jax.experimental.pallas import tpu as pltpu
from jax.experimental.pallas import tpu_sc as plsc

NC = 2
NS = 16
NW = NC * NS
LANES = 16
EP = 128
CH = 128


@functools.lru_cache(maxsize=None)
def _build(B, L, V, E):
    N = B * L
    assert N % (NW * CH) == 0 and E % LANES == 0
    per_w = N // NW
    assert per_w % L == 0
    g_total = per_w // CH

    mesh = plsc.VectorSubcoreMesh(
        core_axis_name="c", subcore_axis_name="s", num_cores=NC, num_subcores=NS
    )

    def body(x_hbm, tok_hbm, pos_hbm, out_hbm,
             idx_v, pos_v, gb0, gb1, ob0, ob1, gs0, gs1, ss0, ss1):
        gbufs = (gb0, gb1)
        obufs = (ob0, ob1)
        gsems = (gs0, gs1)
        ssems = (ss0, ss1)
        wid = lax.axis_index("s") * NC + lax.axis_index("c")
        base = pl.multiple_of(wid * per_w, CH)

        pltpu.sync_copy(x_hbm.at[pl.ds(base, per_w)], idx_v)
        pltpu.sync_copy(pos_hbm, pos_v.at[pl.ds(0, L * E)])
        pltpu.sync_copy(pos_hbm, pos_v.at[pl.ds(L * E, L * E)])

        def start_gather(g):
            b = g % 2
            idx = idx_v.at[pl.ds(g * CH, CH)]
            return pltpu.async_copy(tok_hbm.at[idx], gbufs[b], gsems[b])

        gathers = {0: start_gather(0), 1: start_gather(1)}
        scatters = {}

        for g in range(g_total):
            b = g % 2
            gathers.pop(g).wait()
            if g >= 2:
                scatters.pop(g - 2).wait()
            phase = (g * CH) % L
            gb, ob = gbufs[b], obufs[b]

            def add_row(i, _, gb=gb, ob=ob, phase=phase):
                for j in range(E // LANES):
                    c = j * LANES
                    ob[i, pl.ds(c, LANES)] = (
                        gb[i, pl.ds(c, LANES)]
                        + pos_v[pl.ds((phase + i) * E + c, LANES)]
                    )
                return 0

            lax.fori_loop(0, CH, add_row, 0)

            if g + 2 < g_total:
                gathers[g + 2] = start_gather(g + 2)
            scatters[g] = pltpu.async_copy(
                obufs[b], out_hbm.at[pl.ds(base + g * CH, CH)], ssems[b]
            )

        scatters.pop(g_total - 2).wait()
        scatters.pop(g_total - 1).wait()

    return pl.kernel(
        body,
        out_type=jax.ShapeDtypeStruct((N, E), jnp.float32),
        mesh=mesh,
        scratch_types=[
            pltpu.VMEM((per_w,), jnp.int32),
            pltpu.VMEM((2 * L * E,), jnp.float32),
            pltpu.VMEM((CH, EP), jnp.float32),
            pltpu.VMEM((CH, EP), jnp.float32),
            pltpu.VMEM((CH, E), jnp.float32),
            pltpu.VMEM((CH, E), jnp.float32),
            pltpu.SemaphoreType.DMA,
            pltpu.SemaphoreType.DMA,
            pltpu.SemaphoreType.DMA,
            pltpu.SemaphoreType.DMA,
        ],
    )


def kernel(x, token_table, pos_table):
    B, L = x.shape
    V, E = token_table.shape
    tok128 = jnp.pad(token_table, ((0, 0), (0, EP - E)))
    k = _build(B, L, V, E)
    out = k(
        x.reshape(B * L).astype(jnp.int32),
        tok128,
        pos_table.reshape(L * E),
    )
    return out.reshape(B, L, E)
